# baseline (speedup 1.0000x reference)
import functools

import jax
import jax.numpy as jnp
from jax import lax
from jax.experimental import pallas as pl
from jax.experimental.pallas import tpu as pltpu
from jax.experimental.pallas import tpu_sc as plsc


def _cast_kernel(w_ref, o_ref):
    o_ref[...] = w_ref[...].astype(jnp.bfloat16)


def _logits_kernel(x_ref, ge_ref, wbf_ref, gw_ref, b_ref, lg_ref, *, Tb):
    ge_bf = ge_ref[...].astype(jnp.bfloat16)
    p = lax.dot_general(ge_bf, wbf_ref[...], (((1,), (1,)), ((), ())),
                        preferred_element_type=jnp.float32)
    h = x_ref[...] + p + b_ref[...]
    lg_t = lax.dot_general(h.astype(jnp.bfloat16),
                           gw_ref[...].astype(jnp.bfloat16),
                           (((1,), (1,)), ((), ())),
                           preferred_element_type=jnp.float32)
    lg = jnp.transpose(lg_t, (1, 0))
    nw = Tb // 128
    lg_ref[...] = jnp.stack([lg[:, k * 128:(k + 1) * 128] for k in range(nw)])


def _make_sc_router(L, E, T):
    LE = L * E
    W_TOK = 128
    NW = T // W_TOK
    mesh = plsc.VectorSubcoreMesh(core_axis_name="c", subcore_axis_name="s")

    @functools.partial(
        pl.kernel, mesh=mesh,
        out_type=[
            jax.ShapeDtypeStruct((NW, 2, L, W_TOK), jnp.float32),
            jax.ShapeDtypeStruct((NW, 2, L, W_TOK), jnp.int32),
        ],
        scratch_types=[
            pltpu.VMEM((LE, W_TOK), jnp.float32),
            pltpu.VMEM((2, L, W_TOK), jnp.float32),
            pltpu.VMEM((2, L, W_TOK), jnp.int32),
        ],
    )
    def sc_router(lg_hbm, w_hbm, i_hbm, lg_v, wv, iv):
        wid = lax.axis_index("s") * 2 + lax.axis_index("c")
        pltpu.sync_copy(lg_hbm.at[wid], lg_v)
        neg_inf = jnp.full((16,), -jnp.inf, jnp.float32)
        for l in range(L):
            for v in range(W_TOK // 16):
                sl = pl.ds(v * 16, 16)
                ev = [lg_v[l * E + e, sl] for e in range(E)]
                top1 = ev[0]
                for e in range(1, E):
                    top1 = jnp.maximum(top1, ev[e])
                i1 = jnp.full((16,), E - 1, jnp.int32)
                for e in range(E - 2, -1, -1):
                    i1 = jnp.where(ev[e] == top1, jnp.full((16,), e, jnp.int32), i1)
                top2 = neg_inf
                for e in range(E):
                    masked = jnp.where(i1 == jnp.full((16,), e, jnp.int32),
                                       neg_inf, ev[e])
                    top2 = jnp.maximum(top2, masked)
                i2 = jnp.full((16,), E - 1, jnp.int32)
                for e in range(E - 2, -1, -1):
                    hit = jnp.logical_and(ev[e] == top2,
                                          i1 != jnp.full((16,), e, jnp.int32))
                    i2 = jnp.where(hit, jnp.full((16,), e, jnp.int32), i2)
                ex = jnp.exp(top2 - top1)
                r = 1.0 / (1.0 + ex)
                wv[0, l, sl] = r
                wv[1, l, sl] = ex * r
                iv[0, l, sl] = i1
                iv[1, l, sl] = i2
        pltpu.sync_copy(wv, w_hbm.at[wid])
        pltpu.sync_copy(iv, i_hbm.at[wid])

    return sc_router


def kernel(x, genre_emb, gate_w, genre_proj_w, genre_proj_b):
    T, D = x.shape
    Lyr, E, _ = gate_w.shape
    LE = Lyr * E

    CB = 512
    wbf = pl.pallas_call(
        _cast_kernel,
        grid=(D // CB,),
        in_specs=[pl.BlockSpec((CB, D), lambda i: (i, 0))],
        out_specs=pl.BlockSpec((CB, D), lambda i: (i, 0)),
        out_shape=jax.ShapeDtypeStruct((D, D), jnp.bfloat16),
    )(genre_proj_w)

    gw2 = gate_w.reshape(LE, D)
    b2 = genre_proj_b.reshape(1, D)

    Tb = 256
    NW = T // 128
    lg3 = pl.pallas_call(
        functools.partial(_logits_kernel, Tb=Tb),
        grid=(T // Tb,),
        in_specs=[
            pl.BlockSpec((Tb, D), lambda i: (i, 0)),
            pl.BlockSpec((Tb, D), lambda i: (i, 0)),
            pl.BlockSpec((D, D), lambda i: (0, 0)),
            pl.BlockSpec((LE, D), lambda i: (0, 0)),
            pl.BlockSpec((1, D), lambda i: (0, 0)),
        ],
        out_specs=pl.BlockSpec((Tb // 128, LE, 128), lambda i: (i, 0, 0)),
        out_shape=jax.ShapeDtypeStruct((NW, LE, 128), jnp.float32),
    )(x, genre_emb, wbf, gw2, b2)

    w4, i4 = _make_sc_router(Lyr, E, T)(lg3)

    routing_weights = jnp.transpose(w4, (2, 0, 3, 1)).reshape(Lyr, T, 2)
    expert_indices = jnp.transpose(i4, (2, 0, 3, 1)).reshape(Lyr, T, 2)
    return routing_weights, expert_indices


# speedup vs baseline: 1.0944x; 1.0944x over previous
import functools

import jax
import jax.numpy as jnp
from jax import lax
from jax.experimental import pallas as pl
from jax.experimental.pallas import tpu as pltpu


def _cast_kernel(w_ref, o_ref):
    o_ref[...] = w_ref[...].astype(jnp.bfloat16)


def _gate_kernel(x_ref, ge_ref, wbf_ref, gw_ref, b_ref, w_out_ref, i_out_ref,
                 *, L, E, Tb):
    ge_bf = ge_ref[...].astype(jnp.bfloat16)
    p = lax.dot_general(ge_bf, wbf_ref[...], (((1,), (1,)), ((), ())),
                        preferred_element_type=jnp.float32)
    h = x_ref[...] + p + b_ref[...]
    lg_t = lax.dot_general(h.astype(jnp.bfloat16),
                           gw_ref[...].astype(jnp.bfloat16),
                           (((1,), (1,)), ((), ())),
                           preferred_element_type=jnp.float32)
    lg = jnp.transpose(lg_t, (1, 0)).reshape(L, E, Tb)

    eiota = lax.broadcasted_iota(jnp.int32, (L, E, Tb), 1)
    top1 = jnp.max(lg, axis=1)
    i1 = jnp.min(jnp.where(lg == top1[:, None, :], eiota, E), axis=1)
    masked = jnp.where(eiota == i1[:, None, :], -jnp.inf, lg)
    top2 = jnp.max(masked, axis=1)
    i2 = jnp.min(jnp.where(masked == top2[:, None, :], eiota, E), axis=1)

    w1 = jax.nn.sigmoid(top1 - top2)
    w2 = jax.nn.sigmoid(top2 - top1)
    w_out_ref[...] = jnp.stack([w1, w2])
    i_out_ref[...] = jnp.stack([i1, i2])


def kernel(x, genre_emb, gate_w, genre_proj_w, genre_proj_b):
    T, D = x.shape
    Lyr, E, _ = gate_w.shape
    LE = Lyr * E

    CB = 1024
    wbf = pl.pallas_call(
        _cast_kernel,
        grid=(D // CB,),
        in_specs=[pl.BlockSpec((CB, D), lambda i: (i, 0))],
        out_specs=pl.BlockSpec((CB, D), lambda i: (i, 0)),
        out_shape=jax.ShapeDtypeStruct((D, D), jnp.bfloat16),
    )(genre_proj_w)

    gw2 = gate_w.reshape(LE, D)
    b2 = genre_proj_b.reshape(1, D)

    Tb = 256
    w_out, i_out = pl.pallas_call(
        functools.partial(_gate_kernel, L=Lyr, E=E, Tb=Tb),
        grid=(T // Tb,),
        in_specs=[
            pl.BlockSpec((Tb, D), lambda i: (i, 0)),
            pl.BlockSpec((Tb, D), lambda i: (i, 0)),
            pl.BlockSpec((D, D), lambda i: (0, 0)),
            pl.BlockSpec((LE, D), lambda i: (0, 0)),
            pl.BlockSpec((1, D), lambda i: (0, 0)),
        ],
        out_specs=[
            pl.BlockSpec((2, Lyr, Tb), lambda i: (0, 0, i)),
            pl.BlockSpec((2, Lyr, Tb), lambda i: (0, 0, i)),
        ],
        out_shape=[
            jax.ShapeDtypeStruct((2, Lyr, T), jnp.float32),
            jax.ShapeDtypeStruct((2, Lyr, T), jnp.int32),
        ],
    )(x, genre_emb, wbf, gw2, b2)

    routing_weights = jnp.transpose(w_out, (1, 2, 0))
    expert_indices = jnp.transpose(i_out, (1, 2, 0))
    return routing_weights, expert_indices


# speedup vs baseline: 1.1104x; 1.0146x over previous
import functools

import jax
import jax.numpy as jnp
from jax import lax
from jax.experimental import pallas as pl
from jax.experimental.pallas import tpu as pltpu


def _cast_kernel(w_ref, o_ref):
    o_ref[...] = w_ref[...].astype(jnp.bfloat16)


def _gate_kernel(x_ref, ge_ref, wbf_ref, gw_ref, b_ref, w_out_ref, i_out_ref,
                 *, L, E, Tb):
    ge_bf = ge_ref[...].astype(jnp.bfloat16)
    p = lax.dot_general(ge_bf, wbf_ref[...], (((1,), (1,)), ((), ())),
                        preferred_element_type=jnp.float32)
    h = x_ref[...] + p + b_ref[...]
    lg_t = lax.dot_general(h.astype(jnp.bfloat16),
                           gw_ref[...].astype(jnp.bfloat16),
                           (((1,), (1,)), ((), ())),
                           preferred_element_type=jnp.float32)
    lg = jnp.transpose(lg_t, (1, 0)).reshape(L, E, Tb)

    eiota = lax.broadcasted_iota(jnp.int32, (L, E, Tb), 1)
    top1 = jnp.max(lg, axis=1)
    i1 = jnp.min(jnp.where(lg == top1[:, None, :], eiota, E), axis=1)
    masked = jnp.where(eiota == i1[:, None, :], -jnp.inf, lg)
    top2 = jnp.max(masked, axis=1)
    i2 = jnp.min(jnp.where(masked == top2[:, None, :], eiota, E), axis=1)

    w1 = jax.nn.sigmoid(top1 - top2)
    w2 = jax.nn.sigmoid(top2 - top1)
    w_out_ref[...] = jnp.stack([w1, w2])
    i_out_ref[...] = jnp.stack([i1, i2])


def kernel(x, genre_emb, gate_w, genre_proj_w, genre_proj_b):
    T, D = x.shape
    Lyr, E, _ = gate_w.shape
    LE = Lyr * E

    CB = 512
    wbf = pl.pallas_call(
        _cast_kernel,
        grid=(D // CB,),
        in_specs=[pl.BlockSpec((CB, D), lambda i: (i, 0))],
        out_specs=pl.BlockSpec((CB, D), lambda i: (i, 0)),
        out_shape=jax.ShapeDtypeStruct((D, D), jnp.bfloat16),
    )(genre_proj_w)

    gw2 = gate_w.reshape(LE, D)
    b2 = genre_proj_b.reshape(1, D)

    Tb = 256
    w_out, i_out = pl.pallas_call(
        functools.partial(_gate_kernel, L=Lyr, E=E, Tb=Tb),
        grid=(T // Tb,),
        in_specs=[
            pl.BlockSpec((Tb, D), lambda i: (i, 0)),
            pl.BlockSpec((Tb, D), lambda i: (i, 0)),
            pl.BlockSpec((D, D), lambda i: (0, 0)),
            pl.BlockSpec((LE, D), lambda i: (0, 0)),
            pl.BlockSpec((1, D), lambda i: (0, 0)),
        ],
        out_specs=[
            pl.BlockSpec((2, Lyr, Tb), lambda i: (0, 0, i)),
            pl.BlockSpec((2, Lyr, Tb), lambda i: (0, 0, i)),
        ],
        out_shape=[
            jax.ShapeDtypeStruct((2, Lyr, T), jnp.float32),
            jax.ShapeDtypeStruct((2, Lyr, T), jnp.int32),
        ],
    )(x, genre_emb, wbf, gw2, b2)

    routing_weights = jnp.transpose(w_out, (1, 2, 0))
    expert_indices = jnp.transpose(i_out, (1, 2, 0))
    return routing_weights, expert_indices
